# Initial kernel scaffold; baseline (speedup 1.0000x reference)
import jax
import jax.numpy as jnp
from jax.experimental import pallas as pl


def kernel(xyz, norm, fps_idx, knn_idx, W1, b1, W2, b2):
    raise NotImplementedError("write your pallas kernel here")


# speedup vs baseline: 42.2300x; 42.2300x over previous
import functools

import jax
import jax.numpy as jnp
from jax import lax
from jax.experimental import pallas as pl
from jax.experimental.pallas import tpu as pltpu
from jax.experimental.pallas import tpu_sc as plsc

_BS = 128
_ROW = 16


def _cross(ax, ay, az, bx, by, bz):
    return ay * bz - az * by, az * bx - ax * bz, ax * by - ay * bx


def _tc_body(gt_ref, nt_ref, w1a_ref, w2a_ref, out_ref):
    f32 = jnp.float32
    eps = 1e-07
    g = gt_ref[0]
    K = g.shape[1]
    BS = g.shape[2]
    gx, gy, gz = g[0], g[1], g[2]
    gnx, gny, gnz = g[3], g[4], g[5]
    c = nt_ref[0]
    cx, cy, cz = c[0:1, :], c[1:2, :], c[2:3, :]
    nx, ny, nz = c[3:4, :], c[4:5, :], c[5:6, :]

    def rb(x):
        return x.astype(jnp.bfloat16).astype(f32)

    lx, ly, lz = gx - cx, gy - cy, gz - cz
    bnx, bny, bnz = rb(nx), rb(ny), rb(nz)
    dp = rb(lx) * bnx + rb(ly) * bny + rb(lz) * bnz
    px, py, pz = lx - dp * nx, ly - dp * ny, lz - dp * nz
    plen = jnp.sqrt(px * px + py * py + pz * pz)
    ux, uy, uz = px / plen, py / plen, pz / plen
    ux = jnp.where(jnp.isnan(ux), 0.0, ux)
    uy = jnp.where(jnp.isnan(uy), 0.0, uy)
    uz = jnp.where(jnp.isnan(uz), 0.0, uz)
    ki = lax.broadcasted_iota(jnp.int32, (K, BS), 0)
    kif = ki.astype(f32)
    m = jnp.max(plen, axis=0, keepdims=True)
    first = jnp.min(jnp.where(plen == m, kif, float(K)), axis=0, keepdims=True)
    selm = kif == first
    vrx = jnp.sum(jnp.where(selm, ux, 0.0), axis=0, keepdims=True)
    vry = jnp.sum(jnp.where(selm, uy, 0.0), axis=0, keepdims=True)
    vrz = jnp.sum(jnp.where(selm, uz, 0.0), axis=0, keepdims=True)
    dots = rb(ux) * rb(vrx) + rb(uy) * rb(vry) + rb(uz) * rb(vrz)
    sx, sy, sz = _cross(ux, uy, uz, vrx, vry, vrz)
    t = rb(sx) * bnx + rb(sy) * bny + rb(sz) * bnz
    sg = jnp.sign(t)
    sg = jnp.where(ki == 0, 1.0, sg)
    d = sg * dots - (1.0 - sg)

    jio = lax.broadcasted_iota(jnp.int32, (K, K, BS), 0)
    iio = lax.broadcasted_iota(jnp.int32, (K, K, BS), 1)
    d_j = d[:, None, :]
    d_i = d[None, :, :]
    takes = (d_j > d_i) | ((d_j == d_i) & (jio < iio))
    rank = jnp.sum(takes.astype(jnp.int32), axis=0)

    perm = (rank[:, None, :] == iio).astype(f32)

    def permute(v):
        return jnp.sum(perm * v[:, None, :], axis=0)

    xix, xiy, xiz = permute(lx), permute(ly), permute(lz)
    xinx, xiny, xinz = permute(gnx), permute(gny), permute(gnz)

    x3x, x3y, x3z = (jnp.roll(v, 2, axis=0) for v in (xix, xiy, xiz))
    x3nx, x3ny, x3nz = (jnp.roll(v, 2, axis=0) for v in (xinx, xiny, xinz))
    x4x, x4y, x4z = (jnp.roll(v, -2, axis=0) for v in (xix, xiy, xiz))
    x4nx, x4ny, x4nz = (jnp.roll(v, -2, axis=0) for v in (xinx, xiny, xinz))

    l0 = jnp.sqrt(xix * xix + xiy * xiy + xiz * xiz)
    i0 = 1.0 / (l0 + eps)
    u0x, u0y, u0z = -xix * i0, -xiy * i0, -xiz * i0
    s10 = -(u0x * nx + u0y * ny + u0z * nz)
    s20 = u0x * xinx + u0y * xiny + u0z * xinz

    l1 = jnp.sqrt(x3x * x3x + x3y * x3y + x3z * x3z)
    i1 = 1.0 / (l1 + eps)
    u1x, u1y, u1z = -x3x * i1, -x3y * i1, -x3z * i1
    s11 = -(u1x * nx + u1y * ny + u1z * nz)
    s21 = u1x * x3nx + u1y * x3ny + u1z * x3nz

    o2x, o2y, o2z = xix - x3x, xiy - x3y, xiz - x3z
    l2 = jnp.sqrt(o2x * o2x + o2y * o2y + o2z * o2z)
    i2 = 1.0 / (l2 + eps)
    u2x, u2y, u2z = o2x * i2, o2y * i2, o2z * i2
    s12 = -(u2x * xinx + u2y * xiny + u2z * xinz)
    s22 = u2x * x3nx + u2y * x3ny + u2z * x3nz

    angle_0 = u0x * u1x + u0y * u1y + u0z * u1z
    angle_1 = u1x * u2x + u1y * u2y + u1z * u2z

    l4 = jnp.sqrt(x4x * x4x + x4y * x4y + x4z * x4z)
    i4 = 1.0 / (l4 + eps)
    u4x, u4y, u4z = x4x * i4, x4y * i4, x4z * i4
    o5x, o5y, o5z = x4x - xix, x4y - xiy, x4z - xiz
    l5 = jnp.sqrt(o5x * o5x + o5y * o5y + o5z * o5z)
    i5 = 1.0 / (l5 + eps)
    u5x, u5y, u5z = o5x * i5, o5y * i5, o5z * i5
    c40x, c40y, c40z = _cross(u4x, u4y, u4z, u0x, u0y, u0z)
    c10x, c10y, c10z = _cross(u1x, u1y, u1z, u0x, u0y, u0z)
    a11 = -(u4x * u0x + u4y * u0y + u4z * u0z)
    a12 = u4x * u5x + u4y * u5y + u4z * u5z
    a21 = u5x * x4nx + u5y * x4ny + u5z * x4nz
    a22 = -(u0x * x4nx + u0y * x4ny + u0z * x4nz)
    a3 = -(c40x * c10x + c40y * c10y + c40z * c10z)

    ones = jnp.ones((K, BS), f32)
    F = jnp.stack([l0, s10, s20, angle_0, s11, s21, angle_1, s12, s22,
                   a11, a12, a21, a22, a3, ones], axis=0)
    F = F.reshape(15, K * BS)

    bf16 = jnp.bfloat16
    h1 = jnp.maximum(
        jnp.dot(w1a_ref[...].astype(bf16), F.astype(bf16),
                preferred_element_type=f32), 0.0)
    h1 = jnp.concatenate([h1, jnp.ones((1, K * BS), f32)], axis=0)
    h2 = jnp.maximum(
        jnp.dot(w2a_ref[...].astype(bf16), h1.astype(bf16),
                preferred_element_type=f32), 0.0)

    acc = h2[:, 0:BS]
    for k in range(1, K):
        acc = jnp.maximum(acc, h2[:, k * BS:(k + 1) * BS])
    out_ref[0] = acc


def _sc_gather(table, idx3, total_rows):
    nw, ch, _ = idx3.shape
    mesh = plsc.VectorSubcoreMesh(core_axis_name="c", subcore_axis_name="s")
    info = plsc.get_sparse_core_info()
    nc = info.num_cores

    @functools.partial(
        pl.kernel, mesh=mesh,
        compiler_params=pltpu.CompilerParams(use_tc_tiling_on_sc=False),
        out_type=jax.ShapeDtypeStruct((nw * ch * 128, _ROW), jnp.float32),
        scratch_types=[
            pltpu.VMEM((ch, 128), jnp.int32),
            pltpu.VMEM((128, _ROW), jnp.float32),
            pltpu.SemaphoreType.DMA,
        ],
    )
    def run(table_hbm, idx_hbm, out_hbm, idx_v, rows_v, sem):
        wid = lax.axis_index("s") * nc + lax.axis_index("c")
        pltpu.sync_copy(idx_hbm.at[wid], idx_v)

        def body(i, carry):
            pltpu.async_copy(table_hbm.at[idx_v.at[i]], rows_v, sem).wait()
            pltpu.sync_copy(
                rows_v, out_hbm.at[pl.ds((wid * ch + i) * 128, 128)])
            return carry

        lax.fori_loop(0, ch, body, 0)

    return run(table, idx3)


def kernel(xyz, norm, fps_idx, knn_idx, W1, b1, W2, b2):
    f32 = jnp.float32
    B, N, _ = xyz.shape
    S = fps_idx.shape[1]
    K = knn_idx.shape[2]

    table = jnp.concatenate(
        [xyz.astype(f32), norm.astype(f32),
         jnp.zeros((B, N, _ROW - 6), f32)], axis=-1).reshape(B * N, _ROW)
    boff = jnp.arange(B, dtype=jnp.int32) * N
    knn_f = (jnp.transpose(knn_idx.astype(jnp.int32), (0, 2, 1))
             + boff[:, None, None]).reshape(-1)
    fps_f = (fps_idx.astype(jnp.int32) + boff[:, None]).reshape(-1)
    idx_all = jnp.concatenate([knn_f, fps_f])
    total = idx_all.shape[0]

    info = plsc.get_sparse_core_info()
    nw = info.num_cores * info.num_subcores
    per = -(-total // (nw * 128)) * 128
    pad = nw * per - total
    if pad:
        idx_all = jnp.concatenate(
            [idx_all, jnp.zeros((pad,), jnp.int32)])
    rows = _sc_gather(table, idx_all.reshape(nw, per // 128, 128), total)

    g = rows[:B * K * S].reshape(B, K, S, _ROW)
    nf = rows[B * K * S:total].reshape(B, S, _ROW)
    gt = jnp.transpose(g, (0, 3, 1, 2))
    nt = jnp.transpose(nf, (0, 2, 1))
    w1a = jnp.concatenate([W1.T.astype(f32), b1.astype(f32)[:, None]], axis=1)
    w2a = jnp.concatenate([W2.T.astype(f32), b2.astype(f32)[:, None]], axis=1)

    tc_out = pl.pallas_call(
        _tc_body,
        grid=(B, S // _BS),
        in_specs=[
            pl.BlockSpec((1, _ROW, K, _BS), lambda b, s: (b, 0, 0, s)),
            pl.BlockSpec((1, _ROW, _BS), lambda b, s: (b, 0, s)),
            pl.BlockSpec((32, 15), lambda b, s: (0, 0)),
            pl.BlockSpec((64, 33), lambda b, s: (0, 0)),
        ],
        out_specs=pl.BlockSpec((1, 64, _BS), lambda b, s: (b, 0, s)),
        out_shape=jax.ShapeDtypeStruct((B, 64, S), f32),
    )(gt, nt, w1a, w2a)

    new_xyz = nf[:, :, 0:3]
    new_norm = nf[:, :, 3:6]
    new_points = jnp.transpose(tc_out, (0, 2, 1))
    return new_xyz, new_norm, new_points


# speedup vs baseline: 42.2331x; 1.0001x over previous
import functools

import jax
import jax.numpy as jnp
from jax import lax
from jax.experimental import pallas as pl
from jax.experimental.pallas import tpu as pltpu
from jax.experimental.pallas import tpu_sc as plsc

_BS = 128
_ROW = 16


def _cross(ax, ay, az, bx, by, bz):
    return ay * bz - az * by, az * bx - ax * bz, ax * by - ay * bx


def _tc_body(gt_ref, nt_ref, w1a_ref, w2a_ref, out_ref):
    f32 = jnp.float32
    eps = 1e-07
    g = gt_ref[0]
    K = g.shape[1]
    BS = g.shape[2]
    gx, gy, gz = g[0], g[1], g[2]
    gnx, gny, gnz = g[3], g[4], g[5]
    c = nt_ref[0]
    cx, cy, cz = c[0:1, :], c[1:2, :], c[2:3, :]
    nx, ny, nz = c[3:4, :], c[4:5, :], c[5:6, :]

    def rb(x):
        return x.astype(jnp.bfloat16).astype(f32)

    lx, ly, lz = gx - cx, gy - cy, gz - cz
    bnx, bny, bnz = rb(nx), rb(ny), rb(nz)
    dp = rb(lx) * bnx + rb(ly) * bny + rb(lz) * bnz
    px, py, pz = lx - dp * nx, ly - dp * ny, lz - dp * nz
    plen = jnp.sqrt(px * px + py * py + pz * pz)
    ux, uy, uz = px / plen, py / plen, pz / plen
    ux = jnp.where(jnp.isnan(ux), 0.0, ux)
    uy = jnp.where(jnp.isnan(uy), 0.0, uy)
    uz = jnp.where(jnp.isnan(uz), 0.0, uz)
    ki = lax.broadcasted_iota(jnp.int32, (K, BS), 0)
    kif = ki.astype(f32)
    m = jnp.max(plen, axis=0, keepdims=True)
    first = jnp.min(jnp.where(plen == m, kif, float(K)), axis=0, keepdims=True)
    selm = kif == first
    vrx = jnp.sum(jnp.where(selm, ux, 0.0), axis=0, keepdims=True)
    vry = jnp.sum(jnp.where(selm, uy, 0.0), axis=0, keepdims=True)
    vrz = jnp.sum(jnp.where(selm, uz, 0.0), axis=0, keepdims=True)
    dots = rb(ux) * rb(vrx) + rb(uy) * rb(vry) + rb(uz) * rb(vrz)
    sx, sy, sz = _cross(ux, uy, uz, vrx, vry, vrz)
    t = rb(sx) * bnx + rb(sy) * bny + rb(sz) * bnz
    sg = jnp.sign(t)
    sg = jnp.where(ki == 0, 1.0, sg)
    d = sg * dots - (1.0 - sg)

    jio = lax.broadcasted_iota(jnp.int32, (K, K, BS), 0)
    iio = lax.broadcasted_iota(jnp.int32, (K, K, BS), 1)
    d_j = d[:, None, :]
    d_i = d[None, :, :]
    takes = (d_j > d_i) | ((d_j == d_i) & (jio < iio))
    rank = jnp.sum(takes.astype(jnp.int32), axis=0)

    perm = (rank[:, None, :] == iio).astype(f32)

    def permute(v):
        return jnp.sum(perm * v[:, None, :], axis=0)

    xix, xiy, xiz = permute(lx), permute(ly), permute(lz)
    xinx, xiny, xinz = permute(gnx), permute(gny), permute(gnz)

    x3x, x3y, x3z = (jnp.roll(v, 2, axis=0) for v in (xix, xiy, xiz))
    x3nx, x3ny, x3nz = (jnp.roll(v, 2, axis=0) for v in (xinx, xiny, xinz))
    x4x, x4y, x4z = (jnp.roll(v, -2, axis=0) for v in (xix, xiy, xiz))
    x4nx, x4ny, x4nz = (jnp.roll(v, -2, axis=0) for v in (xinx, xiny, xinz))

    l0 = jnp.sqrt(xix * xix + xiy * xiy + xiz * xiz)
    i0 = 1.0 / (l0 + eps)
    u0x, u0y, u0z = -xix * i0, -xiy * i0, -xiz * i0
    s10 = -(u0x * nx + u0y * ny + u0z * nz)
    s20 = u0x * xinx + u0y * xiny + u0z * xinz

    l1 = jnp.sqrt(x3x * x3x + x3y * x3y + x3z * x3z)
    i1 = 1.0 / (l1 + eps)
    u1x, u1y, u1z = -x3x * i1, -x3y * i1, -x3z * i1
    s11 = -(u1x * nx + u1y * ny + u1z * nz)
    s21 = u1x * x3nx + u1y * x3ny + u1z * x3nz

    o2x, o2y, o2z = xix - x3x, xiy - x3y, xiz - x3z
    l2 = jnp.sqrt(o2x * o2x + o2y * o2y + o2z * o2z)
    i2 = 1.0 / (l2 + eps)
    u2x, u2y, u2z = o2x * i2, o2y * i2, o2z * i2
    s12 = -(u2x * xinx + u2y * xiny + u2z * xinz)
    s22 = u2x * x3nx + u2y * x3ny + u2z * x3nz

    angle_0 = u0x * u1x + u0y * u1y + u0z * u1z
    angle_1 = u1x * u2x + u1y * u2y + u1z * u2z

    l4 = jnp.sqrt(x4x * x4x + x4y * x4y + x4z * x4z)
    i4 = 1.0 / (l4 + eps)
    u4x, u4y, u4z = x4x * i4, x4y * i4, x4z * i4
    o5x, o5y, o5z = x4x - xix, x4y - xiy, x4z - xiz
    l5 = jnp.sqrt(o5x * o5x + o5y * o5y + o5z * o5z)
    i5 = 1.0 / (l5 + eps)
    u5x, u5y, u5z = o5x * i5, o5y * i5, o5z * i5
    c40x, c40y, c40z = _cross(u4x, u4y, u4z, u0x, u0y, u0z)
    c10x, c10y, c10z = _cross(u1x, u1y, u1z, u0x, u0y, u0z)
    a11 = -(u4x * u0x + u4y * u0y + u4z * u0z)
    a12 = u4x * u5x + u4y * u5y + u4z * u5z
    a21 = u5x * x4nx + u5y * x4ny + u5z * x4nz
    a22 = -(u0x * x4nx + u0y * x4ny + u0z * x4nz)
    a3 = -(c40x * c10x + c40y * c10y + c40z * c10z)

    ones = jnp.ones((K, BS), f32)
    F = jnp.stack([l0, s10, s20, angle_0, s11, s21, angle_1, s12, s22,
                   a11, a12, a21, a22, a3, ones], axis=0)
    F = F.reshape(15, K * BS)

    bf16 = jnp.bfloat16
    h1 = jnp.maximum(
        jnp.dot(w1a_ref[...].astype(bf16), F.astype(bf16),
                preferred_element_type=f32), 0.0)
    h1 = jnp.concatenate([h1, jnp.ones((1, K * BS), f32)], axis=0)
    h2 = jnp.maximum(
        jnp.dot(w2a_ref[...].astype(bf16), h1.astype(bf16),
                preferred_element_type=f32), 0.0)

    acc = h2[:, 0:BS]
    for k in range(1, K):
        acc = jnp.maximum(acc, h2[:, k * BS:(k + 1) * BS])
    out_ref[0] = acc


def _sc_gather(table, idx3, total_rows):
    nw, ch, _ = idx3.shape
    mesh = plsc.VectorSubcoreMesh(core_axis_name="c", subcore_axis_name="s")
    info = plsc.get_sparse_core_info()
    nc = info.num_cores

    @functools.partial(
        pl.kernel, mesh=mesh,
        compiler_params=pltpu.CompilerParams(use_tc_tiling_on_sc=False),
        out_type=jax.ShapeDtypeStruct((nw * ch * 128, _ROW), jnp.float32),
        scratch_types=[
            pltpu.VMEM((ch, 128), jnp.int32),
            pltpu.VMEM((128, _ROW), jnp.float32),
            pltpu.SemaphoreType.DMA,
        ],
    )
    def run(table_hbm, idx_hbm, out_hbm, idx_v, rows_v, sem):
        wid = lax.axis_index("s") * nc + lax.axis_index("c")
        pltpu.sync_copy(idx_hbm.at[wid], idx_v)

        def body(i, carry):
            pltpu.async_copy(table_hbm.at[idx_v.at[i]], rows_v, sem).wait()
            pltpu.sync_copy(
                rows_v, out_hbm.at[pl.ds((wid * ch + i) * 128, 128)])
            return carry

        lax.fori_loop(0, ch, body, 0)

    return run(table, idx3)


def kernel(xyz, norm, fps_idx, knn_idx, W1, b1, W2, b2):
    f32 = jnp.float32
    B, N, _ = xyz.shape
    S = fps_idx.shape[1]
    K = knn_idx.shape[2]

    table = jnp.concatenate(
        [xyz.astype(f32), norm.astype(f32),
         jnp.zeros((B, N, _ROW - 6), f32)], axis=-1).reshape(B * N, _ROW)
    boff = jnp.arange(B, dtype=jnp.int32) * N
    knn_f = (jnp.transpose(knn_idx.astype(jnp.int32), (0, 2, 1))
             + boff[:, None, None]).reshape(-1)
    fps_f = (fps_idx.astype(jnp.int32) + boff[:, None]).reshape(-1)
    idx_all = jnp.concatenate([knn_f, fps_f])
    total = idx_all.shape[0]

    info = plsc.get_sparse_core_info()
    nw = info.num_cores * info.num_subcores
    per = -(-total // (nw * 128)) * 128
    pad = nw * per - total
    if pad:
        idx_all = jnp.concatenate(
            [idx_all, jnp.zeros((pad,), jnp.int32)])
    rows = _sc_gather(table, idx_all.reshape(nw, per // 128, 128), total)

    g = rows[:B * K * S].reshape(B, K, S, _ROW)
    nf = rows[B * K * S:total].reshape(B, S, _ROW)
    gt = jnp.transpose(g, (0, 3, 1, 2))
    nt = jnp.transpose(nf, (0, 2, 1))
    w1a = jnp.concatenate([W1.T.astype(f32), b1.astype(f32)[:, None]], axis=1)
    w2a = jnp.concatenate([W2.T.astype(f32), b2.astype(f32)[:, None]], axis=1)

    tc_out = pl.pallas_call(
        _tc_body,
        grid=(B, S // _BS),
        in_specs=[
            pl.BlockSpec((1, _ROW, K, _BS), lambda b, s: (b, 0, 0, s)),
            pl.BlockSpec((1, _ROW, _BS), lambda b, s: (b, 0, s)),
            pl.BlockSpec((32, 15), lambda b, s: (0, 0)),
            pl.BlockSpec((64, 33), lambda b, s: (0, 0)),
        ],
        out_specs=pl.BlockSpec((1, 64, _BS), lambda b, s: (b, 0, s)),
        out_shape=jax.ShapeDtypeStruct((B, 64, S), f32),
        compiler_params=pltpu.CompilerParams(
            dimension_semantics=("parallel", "parallel")),
    )(gt, nt, w1a, w2a)

    new_xyz = nf[:, :, 0:3]
    new_norm = nf[:, :, 3:6]
    new_points = jnp.transpose(tc_out, (0, 2, 1))
    return new_xyz, new_norm, new_points


# speedup vs baseline: 45.0736x; 1.0673x over previous
import functools

import jax
import jax.numpy as jnp
from jax import lax
from jax.experimental import pallas as pl
from jax.experimental.pallas import tpu as pltpu
from jax.experimental.pallas import tpu_sc as plsc

_BS = 128
_ROW = 16


def _cross(ax, ay, az, bx, by, bz):
    return ay * bz - az * by, az * bx - ax * bz, ax * by - ay * bx


def _tc_body(gt_ref, nt_ref, w1a_ref, w2a_ref, out_ref):
    f32 = jnp.float32
    eps = 1e-07
    g = gt_ref[0]
    K = g.shape[1]
    BS = g.shape[2]
    gx, gy, gz = g[0], g[1], g[2]
    gnx, gny, gnz = g[3], g[4], g[5]
    c = nt_ref[0]
    cx, cy, cz = c[0:1, :], c[1:2, :], c[2:3, :]
    nx, ny, nz = c[3:4, :], c[4:5, :], c[5:6, :]

    def rb(x):
        return x.astype(jnp.bfloat16).astype(f32)

    lx, ly, lz = gx - cx, gy - cy, gz - cz
    bnx, bny, bnz = rb(nx), rb(ny), rb(nz)
    dp = rb(lx) * bnx + rb(ly) * bny + rb(lz) * bnz
    px, py, pz = lx - dp * nx, ly - dp * ny, lz - dp * nz
    plen = jnp.sqrt(px * px + py * py + pz * pz)
    ux, uy, uz = px / plen, py / plen, pz / plen
    ux = jnp.where(jnp.isnan(ux), 0.0, ux)
    uy = jnp.where(jnp.isnan(uy), 0.0, uy)
    uz = jnp.where(jnp.isnan(uz), 0.0, uz)
    ki = lax.broadcasted_iota(jnp.int32, (K, BS), 0)
    kif = ki.astype(f32)
    m = jnp.max(plen, axis=0, keepdims=True)
    first = jnp.min(jnp.where(plen == m, kif, float(K)), axis=0, keepdims=True)
    selm = kif == first
    vrx = jnp.sum(jnp.where(selm, ux, 0.0), axis=0, keepdims=True)
    vry = jnp.sum(jnp.where(selm, uy, 0.0), axis=0, keepdims=True)
    vrz = jnp.sum(jnp.where(selm, uz, 0.0), axis=0, keepdims=True)
    dots = rb(ux) * rb(vrx) + rb(uy) * rb(vry) + rb(uz) * rb(vrz)
    sx, sy, sz = _cross(ux, uy, uz, vrx, vry, vrz)
    t = rb(sx) * bnx + rb(sy) * bny + rb(sz) * bnz
    sg = jnp.sign(t)
    sg = jnp.where(ki == 0, 1.0, sg)
    d = sg * dots - (1.0 - sg)

    jio = lax.broadcasted_iota(jnp.int32, (K, K, BS), 0)
    iio = lax.broadcasted_iota(jnp.int32, (K, K, BS), 1)
    d_j = d[:, None, :]
    d_i = d[None, :, :]
    takes = (d_j > d_i) | ((d_j == d_i) & (jio < iio))
    rank = jnp.sum(takes.astype(jnp.int32), axis=0)

    perm = (rank[:, None, :] == iio).astype(f32)

    def permute(v):
        return jnp.sum(perm * v[:, None, :], axis=0)

    xix, xiy, xiz = permute(lx), permute(ly), permute(lz)
    xinx, xiny, xinz = permute(gnx), permute(gny), permute(gnz)

    x3x, x3y, x3z = (jnp.roll(v, 2, axis=0) for v in (xix, xiy, xiz))
    x3nx, x3ny, x3nz = (jnp.roll(v, 2, axis=0) for v in (xinx, xiny, xinz))
    x4x, x4y, x4z = (jnp.roll(v, -2, axis=0) for v in (xix, xiy, xiz))
    x4nx, x4ny, x4nz = (jnp.roll(v, -2, axis=0) for v in (xinx, xiny, xinz))

    l0 = jnp.sqrt(xix * xix + xiy * xiy + xiz * xiz)
    i0 = 1.0 / (l0 + eps)
    u0x, u0y, u0z = -xix * i0, -xiy * i0, -xiz * i0
    s10 = -(u0x * nx + u0y * ny + u0z * nz)
    s20 = u0x * xinx + u0y * xiny + u0z * xinz

    l1 = jnp.sqrt(x3x * x3x + x3y * x3y + x3z * x3z)
    i1 = 1.0 / (l1 + eps)
    u1x, u1y, u1z = -x3x * i1, -x3y * i1, -x3z * i1
    s11 = -(u1x * nx + u1y * ny + u1z * nz)
    s21 = u1x * x3nx + u1y * x3ny + u1z * x3nz

    o2x, o2y, o2z = xix - x3x, xiy - x3y, xiz - x3z
    l2 = jnp.sqrt(o2x * o2x + o2y * o2y + o2z * o2z)
    i2 = 1.0 / (l2 + eps)
    u2x, u2y, u2z = o2x * i2, o2y * i2, o2z * i2
    s12 = -(u2x * xinx + u2y * xiny + u2z * xinz)
    s22 = u2x * x3nx + u2y * x3ny + u2z * x3nz

    angle_0 = u0x * u1x + u0y * u1y + u0z * u1z
    angle_1 = u1x * u2x + u1y * u2y + u1z * u2z

    l4 = jnp.sqrt(x4x * x4x + x4y * x4y + x4z * x4z)
    i4 = 1.0 / (l4 + eps)
    u4x, u4y, u4z = x4x * i4, x4y * i4, x4z * i4
    o5x, o5y, o5z = x4x - xix, x4y - xiy, x4z - xiz
    l5 = jnp.sqrt(o5x * o5x + o5y * o5y + o5z * o5z)
    i5 = 1.0 / (l5 + eps)
    u5x, u5y, u5z = o5x * i5, o5y * i5, o5z * i5
    c40x, c40y, c40z = _cross(u4x, u4y, u4z, u0x, u0y, u0z)
    c10x, c10y, c10z = _cross(u1x, u1y, u1z, u0x, u0y, u0z)
    a11 = -(u4x * u0x + u4y * u0y + u4z * u0z)
    a12 = u4x * u5x + u4y * u5y + u4z * u5z
    a21 = u5x * x4nx + u5y * x4ny + u5z * x4nz
    a22 = -(u0x * x4nx + u0y * x4ny + u0z * x4nz)
    a3 = -(c40x * c10x + c40y * c10y + c40z * c10z)

    ones = jnp.ones((K, BS), f32)
    F = jnp.stack([l0, s10, s20, angle_0, s11, s21, angle_1, s12, s22,
                   a11, a12, a21, a22, a3, ones], axis=0)
    F = F.reshape(15, K * BS)

    bf16 = jnp.bfloat16
    h1 = jnp.maximum(
        jnp.dot(w1a_ref[...].astype(bf16), F.astype(bf16),
                preferred_element_type=f32), 0.0)
    h1 = jnp.concatenate([h1, jnp.ones((1, K * BS), f32)], axis=0)
    h2 = jnp.maximum(
        jnp.dot(w2a_ref[...].astype(bf16), h1.astype(bf16),
                preferred_element_type=f32), 0.0)

    acc = h2[:, 0:BS]
    for k in range(1, K):
        acc = jnp.maximum(acc, h2[:, k * BS:(k + 1) * BS])
    out_ref[0] = acc


def _sc_gather(table, idx3, total_rows):
    nw, ch, _ = idx3.shape
    mesh = plsc.VectorSubcoreMesh(core_axis_name="c", subcore_axis_name="s")
    info = plsc.get_sparse_core_info()
    nc = info.num_cores

    @functools.partial(
        pl.kernel, mesh=mesh,
        compiler_params=pltpu.CompilerParams(use_tc_tiling_on_sc=False),
        out_type=jax.ShapeDtypeStruct((nw * ch * 128, _ROW), jnp.float32),
        scratch_types=[
            pltpu.VMEM((ch, 128), jnp.int32),
            pltpu.VMEM((2, 128, _ROW), jnp.float32),
            pltpu.SemaphoreType.DMA((2,)),
        ],
    )
    def run(table_hbm, idx_hbm, out_hbm, idx_v, rows_v, sem):
        wid = lax.axis_index("s") * nc + lax.axis_index("c")
        pltpu.sync_copy(idx_hbm.at[wid], idx_v)
        pltpu.async_copy(table_hbm.at[idx_v.at[0]], rows_v.at[0], sem.at[0])

        def body(i, carry):
            nxt = i + 1
            slot = lax.rem(i, 2)
            nslot = lax.rem(nxt, 2)

            @pl.when(nxt < ch)
            def _():
                pltpu.async_copy(
                    table_hbm.at[idx_v.at[nxt]], rows_v.at[nslot],
                    sem.at[nslot])

            pltpu.make_async_copy(
                table_hbm.at[idx_v.at[i]], rows_v.at[slot],
                sem.at[slot]).wait()
            pltpu.sync_copy(
                rows_v.at[slot], out_hbm.at[pl.ds((wid * ch + i) * 128, 128)])
            return carry

        lax.fori_loop(0, ch, body, 0)

    return run(table, idx3)


def kernel(xyz, norm, fps_idx, knn_idx, W1, b1, W2, b2):
    f32 = jnp.float32
    B, N, _ = xyz.shape
    S = fps_idx.shape[1]
    K = knn_idx.shape[2]

    table = jnp.concatenate(
        [xyz.astype(f32), norm.astype(f32),
         jnp.zeros((B, N, _ROW - 6), f32)], axis=-1).reshape(B * N, _ROW)
    boff = jnp.arange(B, dtype=jnp.int32) * N
    knn_f = (jnp.transpose(knn_idx.astype(jnp.int32), (0, 2, 1))
             + boff[:, None, None]).reshape(-1)
    fps_f = (fps_idx.astype(jnp.int32) + boff[:, None]).reshape(-1)
    idx_all = jnp.concatenate([knn_f, fps_f])
    total = idx_all.shape[0]

    info = plsc.get_sparse_core_info()
    nw = info.num_cores * info.num_subcores
    per = -(-total // (nw * 128)) * 128
    pad = nw * per - total
    if pad:
        idx_all = jnp.concatenate(
            [idx_all, jnp.zeros((pad,), jnp.int32)])
    rows = _sc_gather(table, idx_all.reshape(nw, per // 128, 128), total)

    g = rows[:B * K * S].reshape(B, K, S, _ROW)
    nf = rows[B * K * S:total].reshape(B, S, _ROW)
    gt = jnp.transpose(g, (0, 3, 1, 2))
    nt = jnp.transpose(nf, (0, 2, 1))
    w1a = jnp.concatenate([W1.T.astype(f32), b1.astype(f32)[:, None]], axis=1)
    w2a = jnp.concatenate([W2.T.astype(f32), b2.astype(f32)[:, None]], axis=1)

    tc_out = pl.pallas_call(
        _tc_body,
        grid=(B, S // _BS),
        in_specs=[
            pl.BlockSpec((1, _ROW, K, _BS), lambda b, s: (b, 0, 0, s)),
            pl.BlockSpec((1, _ROW, _BS), lambda b, s: (b, 0, s)),
            pl.BlockSpec((32, 15), lambda b, s: (0, 0)),
            pl.BlockSpec((64, 33), lambda b, s: (0, 0)),
        ],
        out_specs=pl.BlockSpec((1, 64, _BS), lambda b, s: (b, 0, s)),
        out_shape=jax.ShapeDtypeStruct((B, 64, S), f32),
        compiler_params=pltpu.CompilerParams(
            dimension_semantics=("parallel", "parallel")),
    )(gt, nt, w1a, w2a)

    new_xyz = nf[:, :, 0:3]
    new_norm = nf[:, :, 3:6]
    new_points = jnp.transpose(tc_out, (0, 2, 1))
    return new_xyz, new_norm, new_points
